# Initial kernel scaffold; baseline (speedup 1.0000x reference)
import jax
import jax.numpy as jnp
from jax.experimental import pallas as pl


def kernel(x, edge_index, W1, b1, W2, b2):
    raise NotImplementedError("write your pallas kernel here")


# speedup vs baseline: 19.4377x; 19.4377x over previous
import functools

import jax
import jax.numpy as jnp
from jax import lax
from jax.experimental import pallas as pl
from jax.experimental.pallas import tpu as pltpu
from jax.experimental.pallas import tpu_sc as plsc

N = 10000
E = 320000
D_IN = 128
D_HID = 128
D_OUT = 64

NC = 2
NS = 16
NW = NC * NS

N_PAD = 10240
ROWS_PER_TILE = N_PAD // NS
E2 = E + N
CH = 128
NCHUNK = -(-E2 // (NW * CH))
E_PAD = NW * NCHUNK * CH

BLK = 1024
NBLK = N_PAD // BLK

_mesh = plsc.VectorSubcoreMesh(
    core_axis_name="c", subcore_axis_name="s", num_cores=NC, num_subcores=NS)


def _make_deg_kernel():
    @functools.partial(
        pl.kernel,
        out_type=jax.ShapeDtypeStruct((NC, N_PAD, 16), jnp.float32),
        mesh=_mesh,
        compiler_params=pltpu.CompilerParams(use_tc_tiling_on_sc=False),
        scratch_types=[
            pltpu.VMEM((NCHUNK, CH), jnp.int32),
            pltpu.VMEM((CH, 16), jnp.float32),
            pltpu.VMEM((CH, 16), jnp.float32),
            pltpu.VMEM_SHARED((N_PAD, 16), jnp.float32),
        ],
    )
    def deg_kernel(dst_hbm, out_hbm, dst_v, zeros_v, ones_v, acc_sh):
        c = lax.axis_index("c")
        s = lax.axis_index("s")
        wid = s * NC + c

        pltpu.sync_copy(dst_hbm.at[wid], dst_v)

        def fill(i, _):
            zeros_v[i, :] = jnp.zeros((16,), jnp.float32)
            ones_v[i, :] = jnp.ones((16,), jnp.float32)
            return 0
        lax.fori_loop(0, CH, fill, 0)

        def zero_acc(i, _):
            pltpu.sync_copy(
                zeros_v, acc_sh.at[pl.ds(s * ROWS_PER_TILE + i * CH, CH)])
            return 0
        lax.fori_loop(0, ROWS_PER_TILE // CH, zero_acc, 0)
        plsc.subcore_barrier()

        def body(j, _):
            pltpu.sync_copy(ones_v, acc_sh.at[dst_v.at[j]], add=True)
            return 0
        lax.fori_loop(0, NCHUNK, body, 0)
        plsc.subcore_barrier()

        base = s * ROWS_PER_TILE
        pltpu.sync_copy(acc_sh.at[pl.ds(base, ROWS_PER_TILE)],
                        out_hbm.at[c, pl.ds(base, ROWS_PER_TILE)])

    return deg_kernel


def _make_pass_kernel(d):
    @functools.partial(
        pl.kernel,
        out_type=jax.ShapeDtypeStruct((NC, N_PAD, d), jnp.float32),
        mesh=_mesh,
        compiler_params=pltpu.CompilerParams(use_tc_tiling_on_sc=False),
        scratch_types=[
            pltpu.VMEM((NCHUNK, CH), jnp.int32),
            pltpu.VMEM((NCHUNK, CH), jnp.int32),
            pltpu.VMEM((CH, d), jnp.float32),
            pltpu.VMEM_SHARED((N_PAD, d), jnp.float32),
            pltpu.SemaphoreType.DMA,
        ],
    )
    def pass_kernel(src_hbm, dst_hbm, table_hbm, out_hbm,
                    src_v, dst_v, rows_v, acc_sh, sem):
        c = lax.axis_index("c")
        s = lax.axis_index("s")
        wid = s * NC + c

        pltpu.sync_copy(src_hbm.at[wid], src_v)
        pltpu.sync_copy(dst_hbm.at[wid], dst_v)

        def fill_zero(i, _):
            r = i // (d // 16)
            k = i % (d // 16)
            rows_v[r, pl.ds(k * 16, 16)] = jnp.zeros((16,), jnp.float32)
            return 0
        lax.fori_loop(0, CH * (d // 16), fill_zero, 0)

        def zero_acc(i, _):
            pltpu.sync_copy(
                rows_v, acc_sh.at[pl.ds(s * ROWS_PER_TILE + i * CH, CH)])
            return 0
        lax.fori_loop(0, ROWS_PER_TILE // CH, zero_acc, 0)
        plsc.subcore_barrier()

        def body(j, _):
            pltpu.async_copy(table_hbm.at[src_v.at[j]], rows_v, sem).wait()
            pltpu.sync_copy(rows_v, acc_sh.at[dst_v.at[j]], add=True)
            return 0
        lax.fori_loop(0, NCHUNK, body, 0)
        plsc.subcore_barrier()

        base = s * ROWS_PER_TILE
        pltpu.sync_copy(acc_sh.at[pl.ds(base, ROWS_PER_TILE)],
                        out_hbm.at[c, pl.ds(base, ROWS_PER_TILE)])

    return pass_kernel


def _dinv_from_degp(degp):
    deg = degp[0] + degp[1]
    deg = deg[:, 0:1]
    return jnp.where(deg > 0, lax.rsqrt(deg), 0.0)


def _scale_mm_body(degp_ref, x_ref, w_ref, out_ref):
    dinv = _dinv_from_degp(degp_ref[...])
    h = jnp.dot(x_ref[...], w_ref[...], preferred_element_type=jnp.float32)
    out_ref[...] = h * dinv


def _combine_relu_mm_body(degp_ref, acc_ref, b_ref, w_ref, out_ref):
    dinv = _dinv_from_degp(degp_ref[...])
    agg = acc_ref[0] + acc_ref[1]
    t = jnp.maximum(agg * dinv + b_ref[...], 0.0)
    h = jnp.dot(t, w_ref[...], preferred_element_type=jnp.float32)
    out_ref[...] = h * dinv


def _combine_body(degp_ref, acc_ref, b_ref, out_ref):
    dinv = _dinv_from_degp(degp_ref[...])
    agg = acc_ref[0] + acc_ref[1]
    out_ref[...] = agg * dinv + b_ref[...]


def _degp_spec():
    return pl.BlockSpec((NC, BLK, 16), lambda i: (0, i, 0))


def _rows_spec(d):
    return pl.BlockSpec((BLK, d), lambda i: (i, 0))


def _accp_spec(d):
    return pl.BlockSpec((NC, BLK, d), lambda i: (0, i, 0))


def _full_spec(shape):
    return pl.BlockSpec(shape, lambda i: tuple(0 for _ in shape))


def kernel(x, edge_index, W1, b1, W2, b2):
    src = edge_index[0]
    dst = edge_index[1]
    loop = jnp.arange(N, dtype=src.dtype)
    pad = E_PAD - E2
    src3 = jnp.concatenate(
        [src, loop, jnp.full((pad,), N, src.dtype)]).reshape(NW, NCHUNK, CH)
    dst3 = jnp.concatenate(
        [dst, loop, jnp.full((pad,), N, dst.dtype)]).reshape(NW, NCHUNK, CH)
    x_pad = jnp.pad(x, ((0, N_PAD - N), (0, 0)))
    b1r = b1.reshape(1, D_HID)
    b2r = b2.reshape(1, D_OUT)

    degp = _make_deg_kernel()(dst3)

    hn1 = pl.pallas_call(
        _scale_mm_body,
        grid=(NBLK,),
        in_specs=[_degp_spec(), _rows_spec(D_IN), _full_spec((D_IN, D_HID))],
        out_specs=_rows_spec(D_HID),
        out_shape=jax.ShapeDtypeStruct((N_PAD, D_HID), jnp.float32),
    )(degp, x_pad, W1)

    acc1 = _make_pass_kernel(D_HID)(src3, dst3, hn1)

    hn2 = pl.pallas_call(
        _combine_relu_mm_body,
        grid=(NBLK,),
        in_specs=[_degp_spec(), _accp_spec(D_HID), _full_spec((1, D_HID)),
                  _full_spec((D_HID, D_OUT))],
        out_specs=_rows_spec(D_OUT),
        out_shape=jax.ShapeDtypeStruct((N_PAD, D_OUT), jnp.float32),
    )(degp, acc1, b1r, W2)

    acc2 = _make_pass_kernel(D_OUT)(src3, dst3, hn2)

    zf = pl.pallas_call(
        _combine_body,
        grid=(NBLK,),
        in_specs=[_degp_spec(), _accp_spec(D_OUT), _full_spec((1, D_OUT))],
        out_specs=_rows_spec(D_OUT),
        out_shape=jax.ShapeDtypeStruct((N_PAD, D_OUT), jnp.float32),
    )(degp, acc2, b2r)

    return zf[:N]
